# Initial kernel scaffold; baseline (speedup 1.0000x reference)
import jax
import jax.numpy as jnp
from jax.experimental import pallas as pl


def kernel(src, dst, t, x_pad_simu, t_pad, emb_src, emb_dst):
    raise NotImplementedError("write your pallas kernel here")


# speedup vs baseline: 6.8257x; 6.8257x over previous
import jax
import jax.numpy as jnp
from jax import lax
from jax.experimental import pallas as pl


def _body(t_ref, tp_ref, o_ref):
    tp = tp_ref[...]
    tt = t_ref[...]
    rows, cols = tp.shape
    cond = tp <= tt
    iota = lax.broadcasted_iota(jnp.int32, (rows, cols), 1)
    idx = jnp.where(cond, iota, -1)
    best = jnp.max(idx, axis=1, keepdims=True)
    t_last = jnp.sum(jnp.where(iota == best, tp, 0.0), axis=1, keepdims=True)
    o_ref[...] = jnp.where(best >= 0, jnp.exp(-(tt - t_last)), 0.0)


def kernel(src, dst, t, x_pad_simu, t_pad, emb_src, emb_dst):
    B, L = t_pad.shape
    BR = 256
    t2 = t.reshape(B, 1)
    out = pl.pallas_call(
        _body,
        grid=(B // BR,),
        in_specs=[
            pl.BlockSpec((BR, 1), lambda i: (i, 0)),
            pl.BlockSpec((BR, L), lambda i: (i, 0)),
        ],
        out_specs=pl.BlockSpec((BR, 1), lambda i: (i, 0)),
        out_shape=jax.ShapeDtypeStruct((B, 1), jnp.float32),
    )(t2, t_pad)
    return out.reshape(B)


# speedup vs baseline: 8.0250x; 1.1757x over previous
import functools

import jax
import jax.numpy as jnp
from jax import lax
from jax.experimental import pallas as pl
from jax.experimental.pallas import tpu as pltpu
from jax.experimental.pallas import tpu_sc as plsc

B = 8192
L = 2048
TW = 128
PW = L - TW
NW = 32
RPW = B // NW
SENT = 3.4e38


def _sc_body(t_hbm, tpad_hbm, tails_t_hbm, out_hbm, tt, tvec, obuf, rowbuf):
    wid = lax.axis_index("s") * 2 + lax.axis_index("c")
    base = wid * RPW
    pltpu.sync_copy(tails_t_hbm.at[:, pl.ds(base, RPW)], tt)
    pltpu.sync_copy(t_hbm.at[pl.ds(base, RPW)], tvec)
    lane = lax.iota(jnp.int32, 16)
    zeros = jnp.zeros((16,), jnp.float32)
    neg1 = jnp.full((16,), -1, jnp.int32)

    def fallback(r, rowbase, tb16):
        pltpu.sync_copy(tpad_hbm.at[base + rowbase + r, pl.ds(0, PW)], rowbuf)
        tb = tb16[r]

        def chunk(k, c2):
            bi2, bv2 = c2
            x = rowbuf[pl.ds(k * 16, 16)]
            c = x <= tb
            bi2 = jnp.where(c, jnp.full((16,), k, jnp.int32), bi2)
            bv2 = jnp.where(c, x, bv2)
            return bi2, bv2

        bi2, bv2 = lax.fori_loop(0, PW // 16, chunk, (neg1, zeros))
        g = jnp.where(bi2 >= 0, bi2 * 16 + lane, neg1)
        bv = bv2
        for k in (1, 2, 4, 8):
            og = g[lane ^ k]
            ob = bv[lane ^ k]
            take = og > g
            g = jnp.where(take, og, g)
            bv = jnp.where(take, ob, bv)
        res16 = jnp.where(g >= 0, jnp.exp(-(jnp.full((16,), tb) - bv)), zeros)
        cur = obuf[pl.ds(rowbase, 16)]
        obuf[pl.ds(rowbase, 16)] = jnp.where(lane == r, res16, cur)

    def group_body(q, carry):
        rowbase = q * 16
        tb16 = tvec[pl.ds(rowbase, 16)]

        def col(j, bv):
            x = tt[j, pl.ds(rowbase, 16)]
            return jnp.where(x <= tb16, x, bv)

        bv = lax.fori_loop(0, TW, col, jnp.full((16,), SENT, jnp.float32))
        hit = bv < 1.0e38
        obuf[pl.ds(rowbase, 16)] = jnp.where(
            hit, jnp.exp(-(tb16 - bv)), zeros)

        miss16 = jnp.where(hit, 0, 1)
        mm = miss16
        for k in (1, 2, 4, 8):
            mm = jnp.maximum(mm, mm[lane ^ k])

        @pl.when(mm[0] > 0)
        def _():
            for r in range(16):
                pl.when(miss16[r] > 0)(
                    functools.partial(fallback, r, rowbase, tb16))

        return carry

    lax.fori_loop(0, RPW // 16, group_body, 0)
    pltpu.sync_copy(obuf, out_hbm.at[pl.ds(base, RPW)])


@jax.jit
def _sc_call(t, t_pad, tails_t):
    mesh = plsc.VectorSubcoreMesh(core_axis_name="c", subcore_axis_name="s")
    f = pl.kernel(
        _sc_body,
        mesh=mesh,
        out_type=jax.ShapeDtypeStruct((B,), jnp.float32),
        scratch_types=[
            pltpu.VMEM((TW, RPW), jnp.float32),
            pltpu.VMEM((RPW,), jnp.float32),
            pltpu.VMEM((RPW,), jnp.float32),
            pltpu.VMEM((PW,), jnp.float32),
        ],
    )
    return f(t, t_pad, tails_t)


def kernel(src, dst, t, x_pad_simu, t_pad, emb_src, emb_dst):
    tails_t = jnp.transpose(t_pad[:, PW:])
    return _sc_call(t, t_pad, tails_t)


# speedup vs baseline: 9.2674x; 1.1548x over previous
import functools

import jax
import jax.numpy as jnp
from jax import lax
from jax.experimental import pallas as pl
from jax.experimental.pallas import tpu as pltpu
from jax.experimental.pallas import tpu_sc as plsc

B = 8192
L = 2048
TW = 128
PW = L - TW
NW = 32
RPW = B // NW
SENT = 3.4e38


def _sc_body(t_hbm, tpad_hbm, tails_t_hbm, out_hbm, tt, tvec, obuf, rowbuf):
    wid = lax.axis_index("s") * 2 + lax.axis_index("c")
    base = wid * RPW
    pltpu.sync_copy(tails_t_hbm.at[:, pl.ds(base, RPW)], tt)
    pltpu.sync_copy(t_hbm.at[pl.ds(base, RPW)], tvec)
    lane = lax.iota(jnp.int32, 16)
    zeros = jnp.zeros((16,), jnp.float32)
    neg1 = jnp.full((16,), -1, jnp.int32)

    def fallback(r, rowbase, tb16):
        pltpu.sync_copy(tpad_hbm.at[base + rowbase + r, pl.ds(0, PW)], rowbuf)
        tb = tb16[r]

        def chunk(k, c2):
            bi2, bv2 = c2
            x = rowbuf[pl.ds(k * 16, 16)]
            c = x <= tb
            bi2 = jnp.where(c, jnp.full((16,), k, jnp.int32), bi2)
            bv2 = jnp.where(c, x, bv2)
            return bi2, bv2

        bi2, bv2 = lax.fori_loop(0, PW // 16, chunk, (neg1, zeros))
        g = jnp.where(bi2 >= 0, bi2 * 16 + lane, neg1)
        bv = bv2
        for k in (1, 2, 4, 8):
            og = g[lane ^ k]
            ob = bv[lane ^ k]
            take = og > g
            g = jnp.where(take, og, g)
            bv = jnp.where(take, ob, bv)
        res16 = jnp.where(g >= 0, jnp.exp(-(jnp.full((16,), tb) - bv)), zeros)
        cur = obuf[pl.ds(rowbase, 16)]
        obuf[pl.ds(rowbase, 16)] = jnp.where(lane == r, res16, cur)

    def group_body(q, carry):
        rowbase = q * 16
        tb16 = tvec[pl.ds(rowbase, 16)]

        def col8(jo, bv):
            for ji in range(8):
                x = tt[jo * 8 + ji, pl.ds(rowbase, 16)]
                bv = jnp.where(x <= tb16, x, bv)
            return bv

        bv = lax.fori_loop(0, TW // 8, col8,
                           jnp.full((16,), SENT, jnp.float32))
        hit = bv < 1.0e38
        obuf[pl.ds(rowbase, 16)] = jnp.where(
            hit, jnp.exp(-(tb16 - bv)), zeros)

        miss16 = jnp.where(hit, 0, 1)
        mm = miss16
        for k in (1, 2, 4, 8):
            mm = jnp.maximum(mm, mm[lane ^ k])

        @pl.when(mm[0] > 0)
        def _():
            for r in range(16):
                pl.when(miss16[r] > 0)(
                    functools.partial(fallback, r, rowbase, tb16))

        return carry

    lax.fori_loop(0, RPW // 16, group_body, 0)
    pltpu.sync_copy(obuf, out_hbm.at[pl.ds(base, RPW)])


@jax.jit
def _sc_call(t, t_pad, tails_t):
    mesh = plsc.VectorSubcoreMesh(core_axis_name="c", subcore_axis_name="s")
    f = pl.kernel(
        _sc_body,
        mesh=mesh,
        out_type=jax.ShapeDtypeStruct((B,), jnp.float32),
        scratch_types=[
            pltpu.VMEM((TW, RPW), jnp.float32),
            pltpu.VMEM((RPW,), jnp.float32),
            pltpu.VMEM((RPW,), jnp.float32),
            pltpu.VMEM((PW,), jnp.float32),
        ],
    )
    return f(t, t_pad, tails_t)


def kernel(src, dst, t, x_pad_simu, t_pad, emb_src, emb_dst):
    tails_t = jnp.transpose(t_pad[:, PW:])
    return _sc_call(t, t_pad, tails_t)
